# baseline (speedup 1.0000x reference)
import functools

import jax
import jax.numpy as jnp
from jax import lax
from jax.experimental import pallas as pl
from jax.experimental.pallas import tpu as pltpu
from jax.experimental.pallas import tpu_sc as plsc

VOCAB = 1000000
EMB = 64
B = 16384
L = 200
NCLS = 20

NC = 2
NS = 16
LANES = 16
NW = NC * NS
ROWS_PER_W = B // NW
NVEC = EMB // LANES
RPP = 128
NPH = ROWS_PER_W // RPP
NRING = 4

L2 = 2 * L
CHUNKS = (104, 96, 104, 96)
UNROLL = 8


def _sc_pool(data_flat, table2):
    mesh = plsc.VectorSubcoreMesh(
        core_axis_name="c", subcore_axis_name="s", num_cores=NC, num_subcores=NS
    )

    @functools.partial(
        pl.kernel,
        out_type=jax.ShapeDtypeStruct((B, EMB), jnp.float32),
        mesh=mesh,
        compiler_params=pltpu.CompilerParams(use_tc_tiling_on_sc=False),
        scratch_types=[
            pltpu.VMEM((2, RPP * L), jnp.int32),
            pltpu.VMEM((NRING, L2), jnp.int32),
            pltpu.VMEM((NRING, L2, EMB // 2), jnp.float32),
            pltpu.VMEM((2, RPP, EMB), jnp.float32),
            pltpu.SemaphoreType.DMA,
            pltpu.SemaphoreType.DMA,
            pltpu.SemaphoreType.DMA,
            pltpu.SemaphoreType.DMA,
            pltpu.SemaphoreType.DMA,
            pltpu.SemaphoreType.DMA,
            pltpu.SemaphoreType.DMA,
            pltpu.SemaphoreType.DMA,
        ],
    )
    def k(data_hbm, table_hbm, out_hbm, idxg, xidx, rows, ostage,
          is0, is1, g0, g1, g2, g3, o0, o1):
        isem = (is0, is1)
        gsem = (g0, g1, g2, g3)
        osem = (o0, o1)
        wid = lax.axis_index("s") * NC + lax.axis_index("c")
        base = wid * ROWS_PER_W

        def issue_idx(p, pp):
            return pltpu.async_copy(
                data_hbm.at[pl.ds((base + p * RPP) * L, RPP * L)],
                idxg.at[pp], isem[pp])

        def expand_idx(idx_p, roff, slot):
            dst = xidx.at[slot]
            offs = [16 * kk for kk in range(12)] + [L - 16]
            for off in offs:
                v2 = 2 * idx_p[pl.ds(roff * L + off, 16)]
                dst[pl.ds(off, 16)] = v2
                dst[pl.ds(L + off, 16)] = v2 + 1

        def issue_gathers(slot):
            off = 0
            for c in CHUNKS:
                pltpu.async_copy(
                    table_hbm.at[xidx.at[slot].at[pl.ds(off, c)]],
                    rows.at[slot].at[pl.ds(off, c)], gsem[slot])
                off += c

        def wait_gathers(slot):
            pltpu.make_async_copy(
                table_hbm.at[pl.ds(0, L2)], rows.at[slot], gsem[slot]).wait()

        def accumulate(slot):
            slot_ref = rows.at[slot]

            def body(jj, accs):
                accs = list(accs)
                for u in range(UNROLL):
                    j = jj * UNROLL + u
                    for g in range(2):
                        accs[g] = accs[g] + slot_ref[j, pl.ds(g * LANES, LANES)]
                        accs[2 + g] = accs[2 + g] + slot_ref[L + j, pl.ds(g * LANES, LANES)]
                return tuple(accs)

            accs = tuple(jnp.zeros((LANES,), jnp.float32) for _ in range(NVEC))
            return lax.fori_loop(0, L // UNROLL, body, accs)

        def store_row(opp, r, accs):
            for t in range(NVEC):
                opp[r, pl.ds(t * LANES, LANES)] = accs[t]

        idesc = [issue_idx(0, 0), None]
        odesc = [None, None]
        for p in range(NPH):
            pp = p % 2
            if odesc[pp] is not None:
                odesc[pp].wait()
            idesc[pp].wait()
            if p + 1 < NPH:
                idesc[(p + 1) % 2] = issue_idx(p + 1, (p + 1) % 2)
            idx_p = idxg.at[pp]
            opp = ostage.at[pp]
            for s in range(NRING):
                expand_idx(idx_p, s, s)
                issue_gathers(s)

            def inner(h, carry, idx_p=idx_p, opp=opp):
                for j in range(NRING):
                    r = NRING * h + j
                    wait_gathers(j)
                    expand_idx(idx_p, r + NRING, j)
                    accs = accumulate(j)
                    store_row(opp, r, accs)
                    issue_gathers(j)
                return carry

            lax.fori_loop(0, RPP // NRING - 1, inner, 0)
            for j in range(NRING):
                r = RPP - NRING + j
                wait_gathers(j)
                accs = accumulate(j)
                store_row(opp, r, accs)
            odesc[pp] = pltpu.async_copy(
                opp, out_hbm.at[pl.ds(base + p * RPP, RPP)], osem[pp])
        odesc[0].wait()
        odesc[1].wait()

    return k(data_flat, table2)


def _tc_linear(sums, inv_len, W2, b2):
    BLK = 2048

    def body(s_ref, l_ref, w_ref, b_ref, o_ref):
        pooled = s_ref[...] * l_ref[...]
        o_ref[...] = (
            lax.dot_general(
                pooled, w_ref[...], (((1,), (1,)), ((), ())),
                preferred_element_type=jnp.float32,
            )
            + b_ref[...]
        )

    return pl.pallas_call(
        body,
        grid=(B // BLK,),
        in_specs=[
            pl.BlockSpec((BLK, EMB), lambda i: (i, 0)),
            pl.BlockSpec((BLK, 1), lambda i: (i, 0)),
            pl.BlockSpec((NCLS, EMB), lambda i: (0, 0)),
            pl.BlockSpec((1, NCLS), lambda i: (0, 0)),
        ],
        out_specs=pl.BlockSpec((BLK, NCLS), lambda i: (i, 0)),
        out_shape=jax.ShapeDtypeStruct((B, NCLS), jnp.float32),
    )(sums, inv_len, W2, b2)


def kernel(data, length, embed_table, W, b):
    data_flat = data.reshape(B * L).astype(jnp.int32)
    table2 = embed_table.reshape(2 * VOCAB, EMB // 2)
    sums = _sc_pool(data_flat, table2)
    inv_len = (1.0 / length.astype(jnp.float32)).reshape(B, 1)
    return _tc_linear(sums, inv_len, W, b.reshape(1, NCLS))
